# baseline (speedup 1.0000x reference)
import dataclasses
import functools

import jax
import jax.numpy as jnp
from jax import lax
from jax.experimental import pallas as pl
from jax.experimental.pallas import tpu as pltpu
from jax.experimental.pallas import tpu_sc as plsc

_B = 64
_T = 2048
_D = 768
_LANES = 16
_WORKERS = 32
_ROWS_PER_WORKER = _B // _WORKERS
_CHUNKS = _T // _LANES


def _pooler_kernel(emb_hbm, tok_hbm, out_hbm, tok_v, sem0, sem1):
    wid = lax.axis_index("s") * 2 + lax.axis_index("c")
    lane = lax.iota(jnp.int32, 16)
    b0 = wid * _ROWS_PER_WORKER
    b1 = b0 + 1

    pltpu.async_copy(tok_hbm.at[wid], tok_v, sem0).wait()

    neg = jnp.full((_LANES,), -(2**31), jnp.int32)

    def body(i, carry):
        p0, p1, q0, q1 = carry
        base = i * (2 * _LANES)
        c00 = tok_v[pl.ds(base, _LANES)]
        c01 = tok_v[pl.ds(base + _LANES, _LANES)]
        c10 = tok_v[pl.ds(_T + base, _LANES)]
        c11 = tok_v[pl.ds(_T + base + _LANES, _LANES)]
        r0 = (_T - 1) - base - lane
        r1 = r0 - _LANES
        return (
            jnp.maximum(p0, (c00 << 11) + r0),
            jnp.maximum(p1, (c01 << 11) + r1),
            jnp.maximum(q0, (c10 << 11) + r0),
            jnp.maximum(q1, (c11 << 11) + r1),
        )

    p0, p1, q0, q1 = lax.fori_loop(
        0, _T // (2 * _LANES), body, (neg, neg, neg, neg), unroll=8
    )
    key0 = jnp.max(jnp.maximum(p0, p1))
    key1 = jnp.max(jnp.maximum(q0, q1))
    pos0 = (_T - 1) - (key0 & (_T - 1))
    pos1 = (_T - 1) - (key1 & (_T - 1))
    o0 = pltpu.async_copy(emb_hbm.at[b0, pos0], out_hbm.at[b0], sem0)
    o1 = pltpu.async_copy(emb_hbm.at[b1, pos1], out_hbm.at[b1], sem1)
    o0.wait()
    o1.wait()


def kernel(text_embeddings, token_ids):
    mesh = plsc.VectorSubcoreMesh(core_axis_name="c", subcore_axis_name="s")
    cp = pltpu.CompilerParams()
    if "needs_layout_passes" in pltpu.CompilerParams.__dataclass_fields__:
        cp = dataclasses.replace(cp, needs_layout_passes=False)
    k = functools.partial(
        pl.kernel,
        mesh=mesh,
        out_type=jax.ShapeDtypeStruct((_B, _D), jnp.float32),
        scratch_types=[
            pltpu.VMEM((2 * _T,), jnp.int32),
            pltpu.SemaphoreType.DMA,
            pltpu.SemaphoreType.DMA,
        ],
        compiler_params=cp,
    )(_pooler_kernel)
    tok2 = token_ids.astype(jnp.int32).reshape(_B // 2, 2 * _T)
    return k(text_embeddings, tok2)


# speedup vs baseline: 1.0153x; 1.0153x over previous
import dataclasses
import functools

import jax
import jax.numpy as jnp
from jax import lax
from jax.experimental import pallas as pl
from jax.experimental.pallas import tpu as pltpu
from jax.experimental.pallas import tpu_sc as plsc

_B = 64
_T = 2048
_D = 768
_LANES = 16
_WORKERS = 32
_ROWS_PER_WORKER = _B // _WORKERS


def _row_argmax(tok_v, lane):
    neg = jnp.full((_LANES,), -(2**31), jnp.int32)

    def body(i, carry):
        p0, p1 = carry
        base = i * (2 * _LANES)
        c0 = tok_v[pl.ds(base, _LANES)]
        c1 = tok_v[pl.ds(base + _LANES, _LANES)]
        r0 = (_T - 1) - base - lane
        return (
            jnp.maximum(p0, (c0 << 11) + r0),
            jnp.maximum(p1, (c1 << 11) + (r0 - _LANES)),
        )

    p0, p1 = lax.fori_loop(0, _T // (2 * _LANES), body, (neg, neg), unroll=4)
    best_key = jnp.max(jnp.maximum(p0, p1))
    return (_T - 1) - (best_key & (_T - 1))


def _pooler_kernel(emb_hbm, tok_hbm, out_hbm, tok0_v, tok1_v, si0, si1, so0, so1):
    wid = lax.axis_index("s") * 2 + lax.axis_index("c")
    lane = lax.iota(jnp.int32, 16)
    b0 = wid * _ROWS_PER_WORKER
    b1 = b0 + 1

    cp0 = pltpu.async_copy(tok_hbm.at[b0], tok0_v, si0)
    cp1 = pltpu.async_copy(tok_hbm.at[b1], tok1_v, si1)

    cp0.wait()
    pos0 = _row_argmax(tok0_v, lane)
    o0 = pltpu.async_copy(emb_hbm.at[b0, pos0], out_hbm.at[b0], so0)

    cp1.wait()
    pos1 = _row_argmax(tok1_v, lane)
    o1 = pltpu.async_copy(emb_hbm.at[b1, pos1], out_hbm.at[b1], so1)

    o0.wait()
    o1.wait()


def kernel(text_embeddings, token_ids):
    mesh = plsc.VectorSubcoreMesh(core_axis_name="c", subcore_axis_name="s")
    cp = pltpu.CompilerParams()
    if "needs_layout_passes" in pltpu.CompilerParams.__dataclass_fields__:
        cp = dataclasses.replace(cp, needs_layout_passes=False)
    k = functools.partial(
        pl.kernel,
        mesh=mesh,
        out_type=jax.ShapeDtypeStruct((_B, _D), jnp.float32),
        scratch_types=[
            pltpu.VMEM((_T,), jnp.int32),
            pltpu.VMEM((_T,), jnp.int32),
            pltpu.SemaphoreType.DMA,
            pltpu.SemaphoreType.DMA,
            pltpu.SemaphoreType.DMA,
            pltpu.SemaphoreType.DMA,
        ],
        compiler_params=cp,
    )(_pooler_kernel)
    return k(text_embeddings, token_ids.astype(jnp.int32))
